# Initial kernel scaffold; baseline (speedup 1.0000x reference)
import jax
import jax.numpy as jnp
from jax.experimental import pallas as pl


def kernel(distance_to_center, edge_index, h_gru, gru_inp, beta, W1, b1, W2, b2):
    raise NotImplementedError("write your pallas kernel here")


# speedup vs baseline: 33.5180x; 33.5180x over previous
import functools

import jax
import jax.numpy as jnp
from jax import lax
from jax.experimental import pallas as pl
from jax.experimental.pallas import tpu as pltpu
from jax.experimental.pallas import tpu_sc as plsc

N = 50000
E = 1600000
NTILES = 32
NCORES = 2
COLS = 1568
NPAD = NTILES * COLS
EPT = E // NTILES
CHUNK = 10000
VECS = CHUNK // 16

_mesh = plsc.VectorSubcoreMesh(core_axis_name="c", subcore_axis_name="s")
_sc_params = pltpu.CompilerParams(needs_layout_passes=False)


def _wid():
    return lax.axis_index("s") * NCORES + lax.axis_index("c")


@functools.partial(
    pl.kernel,
    mesh=_mesh,
    out_type=jax.ShapeDtypeStruct((NTILES * NPAD,), jnp.float32),
    compiler_params=_sc_params,
    scratch_types=[
        pltpu.VMEM((N,), jnp.float32),
        pltpu.VMEM((NPAD,), jnp.float32),
        pltpu.VMEM((CHUNK,), jnp.int32),
        pltpu.VMEM((CHUNK,), jnp.int32),
    ],
)
def _segmin_kernel(dist_hbm, src_hbm, dst_hbm, init_hbm, part_hbm,
                   dist_v, table_v, src_v, dst_v):
    wid = _wid()
    pltpu.sync_copy(dist_hbm, dist_v)
    pltpu.sync_copy(init_hbm, table_v)
    base_e = wid * EPT

    def chunk_body(c, carry):
        off = base_e + c * CHUNK
        pltpu.sync_copy(src_hbm.at[pl.ds(off, CHUNK)], src_v)
        pltpu.sync_copy(dst_hbm.at[pl.ds(off, CHUNK)], dst_v)

        def vec_body(j, carry2):
            s16 = src_v[pl.ds(j * 16, 16)]
            d16 = dst_v[pl.ds(j * 16, 16)]
            msg = plsc.load_gather(dist_v, [s16])

            def cond(m):
                return jnp.any(m)

            def body(m):
                cur = plsc.load_gather(table_v, [d16])
                new = jnp.minimum(cur, msg)
                plsc.store_scatter(table_v, [d16], new, mask=m)
                cur2 = plsc.load_gather(table_v, [d16])
                return msg < cur2

            lax.while_loop(cond, body, jnp.ones((16,), jnp.bool_))
            return carry2

        lax.fori_loop(0, VECS, vec_body, 0)
        return carry

    lax.fori_loop(0, EPT // CHUNK, chunk_body, 0)
    pltpu.sync_copy(table_v, part_hbm.at[pl.ds(wid * NPAD, NPAD)])


@functools.partial(
    pl.kernel,
    mesh=_mesh,
    out_type=jax.ShapeDtypeStruct((NPAD,), jnp.float32),
    compiler_params=_sc_params,
    scratch_types=[
        pltpu.VMEM((NTILES * COLS,), jnp.float32),
        pltpu.VMEM((COLS,), jnp.float32),
        pltpu.SemaphoreType.DMA,
    ],
)
def _merge_kernel(part_hbm, md_hbm, rows_v, acc_v, sem):
    wid = _wid()
    base = wid * COLS
    copies = [
        pltpu.async_copy(part_hbm.at[pl.ds(r * NPAD + base, COLS)],
                         rows_v.at[pl.ds(r * COLS, COLS)], sem)
        for r in range(NTILES)
    ]
    for cp in copies:
        cp.wait()

    def col_body(j, carry):
        a = rows_v[pl.ds(j * 16, 16)]
        for r in range(1, NTILES):
            a = jnp.minimum(a, rows_v[pl.ds(r * COLS + j * 16, 16)])
        acc_v[pl.ds(j * 16, 16)] = a
        return carry

    lax.fori_loop(0, COLS // 16, col_body, 0)
    pltpu.sync_copy(acc_v, md_hbm.at[pl.ds(base, COLS)])


ROWS = 1000
GRID = N // ROWS


def _mlp_body(hg_ref, gi_ref, w1a_ref, w1b_ref, b1_ref, w2_ref, b2_ref,
              beta_ref, md_ref, dist_ref, temp_ref, switch_ref, label_ref):
    x = (jnp.dot(hg_ref[...], w1a_ref[...], preferred_element_type=jnp.float32)
         + jnp.dot(gi_ref[...], w1b_ref[...], preferred_element_type=jnp.float32)
         + b1_ref[...])
    h1 = jnp.where(x > 0, x, jnp.exp(x) - 1.0)
    t = jnp.dot(h1, w2_ref[...], preferred_element_type=jnp.float32) + b2_ref[...]
    temp = 1.0 / (1.0 + jnp.exp(-t))
    dw = beta_ref[...] ** 2
    md = jnp.minimum(md_ref[...], 1e4)
    dscore = jnp.exp((-dw) * md)
    switch = dscore * temp
    hard = (switch >= 0.5).astype(jnp.float32)
    label = (hard - switch) + switch
    dist_ref[...] = dscore
    temp_ref[...] = temp
    switch_ref[...] = switch
    label_ref[...] = label


_row_spec = pl.BlockSpec((ROWS, 256), lambda i: (i, 0))
_fix = lambda i: (0, 0)
_col_spec = pl.BlockSpec((ROWS, 1), lambda i: (i, 0))

_mlp_call = pl.pallas_call(
    _mlp_body,
    grid=(GRID,),
    in_specs=[
        _row_spec,
        _row_spec,
        pl.BlockSpec((256, 256), _fix),
        pl.BlockSpec((256, 256), _fix),
        pl.BlockSpec((1, 256), _fix),
        pl.BlockSpec((256, 1), _fix),
        pl.BlockSpec((1, 1), _fix),
        pl.BlockSpec((1, 1), _fix),
        _col_spec,
    ],
    out_specs=[_col_spec, _col_spec, _col_spec, _col_spec],
    out_shape=[jax.ShapeDtypeStruct((N, 1), jnp.float32)] * 4,
)


def kernel(distance_to_center, edge_index, h_gru, gru_inp, beta, W1, b1, W2, b2):
    src = edge_index[0]
    dst = edge_index[1]
    init = jnp.full((NPAD,), 1e4, jnp.float32)
    part = _segmin_kernel(distance_to_center, src, dst, init)
    md_pad = _merge_kernel(part)
    md = md_pad[:N].reshape(N, 1)
    dist_s, temp_s, switch_s, label_s = _mlp_call(
        h_gru, gru_inp, W1[:256], W1[256:], b1.reshape(1, 256),
        W2, b2.reshape(1, 1), beta, md)
    return (dist_s.reshape(N), temp_s.reshape(N),
            switch_s.reshape(N), label_s.reshape(N))


# speedup vs baseline: 71.0348x; 2.1193x over previous
import functools

import jax
import jax.numpy as jnp
from jax import lax
from jax.experimental import pallas as pl
from jax.experimental.pallas import tpu as pltpu
from jax.experimental.pallas import tpu_sc as plsc

N = 50000
E = 1600000
NTILES = 32
NCORES = 2
COLS = 1568
NPAD = NTILES * COLS
EPT = E // NTILES
CHUNK = 2000
NCHUNKS = EPT // CHUNK
G = 5
BATCH = G * 16
NBATCH = CHUNK // BATCH

_mesh = plsc.VectorSubcoreMesh(core_axis_name="c", subcore_axis_name="s")
_sc_params = pltpu.CompilerParams(needs_layout_passes=False)


def _wid():
    return lax.axis_index("s") * NCORES + lax.axis_index("c")


@functools.partial(
    pl.kernel,
    mesh=_mesh,
    out_type=jax.ShapeDtypeStruct((NTILES * NPAD,), jnp.float32),
    compiler_params=_sc_params,
    scratch_types=[
        pltpu.VMEM((N,), jnp.float32),
        pltpu.VMEM((NPAD,), jnp.float32),
        pltpu.VMEM((2 * CHUNK,), jnp.int32),
        pltpu.VMEM((2 * CHUNK,), jnp.int32),
        pltpu.SemaphoreType.DMA,
    ],
)
def _segmin_kernel(dist_hbm, edge_hbm, init_hbm, part_hbm,
                   dist_v, table_v, src_v, dst_v, sem):
    wid = _wid()
    base_e = wid * EPT
    pltpu.async_copy(edge_hbm.at[pl.ds(base_e, CHUNK)],
                     src_v.at[pl.ds(0, CHUNK)], sem)
    pltpu.async_copy(edge_hbm.at[pl.ds(E + base_e, CHUNK)],
                     dst_v.at[pl.ds(0, CHUNK)], sem)
    pltpu.sync_copy(dist_hbm, dist_v)
    pltpu.sync_copy(init_hbm, table_v)

    def chunk_body(c, carry):
        par = lax.rem(c, 2) * CHUNK
        pltpu.make_async_copy(edge_hbm.at[pl.ds(0, CHUNK)],
                              src_v.at[pl.ds(par, CHUNK)], sem).wait()
        pltpu.make_async_copy(edge_hbm.at[pl.ds(0, CHUNK)],
                              dst_v.at[pl.ds(par, CHUNK)], sem).wait()

        nxt = (CHUNK - par)

        @pl.when(c + 1 < NCHUNKS)
        def _():
            off = base_e + (c + 1) * CHUNK
            pltpu.async_copy(edge_hbm.at[pl.ds(off, CHUNK)],
                             src_v.at[pl.ds(nxt, CHUNK)], sem)
            pltpu.async_copy(edge_hbm.at[pl.ds(E + off, CHUNK)],
                             dst_v.at[pl.ds(nxt, CHUNK)], sem)

        def batch_body(j, carry2):
            b0 = par + j * BATCH
            ss = [src_v[pl.ds(b0 + g * 16, 16)] for g in range(G)]
            ds_ = [dst_v[pl.ds(b0 + g * 16, 16)] for g in range(G)]
            ms = [plsc.load_gather(dist_v, [s]) for s in ss]
            curs = [plsc.load_gather(table_v, [d]) for d in ds_]
            for g in range(G):
                plsc.store_scatter(table_v, [ds_[g]],
                                   jnp.minimum(curs[g], ms[g]))
            cur2 = [plsc.load_gather(table_v, [d]) for d in ds_]
            lost = [ms[g] < cur2[g] for g in range(G)]
            anyl = lost[0]
            for g in range(1, G):
                anyl = anyl | lost[g]

            @pl.when(jnp.any(anyl))
            def _():
                def cond(masks):
                    a = masks[0]
                    for g in range(1, G):
                        a = a | masks[g]
                    return jnp.any(a)

                def body(masks):
                    out = []
                    for g in range(G):
                        cur = plsc.load_gather(table_v, [ds_[g]])
                        plsc.store_scatter(table_v, [ds_[g]],
                                           jnp.minimum(cur, ms[g]),
                                           mask=masks[g])
                        chk = plsc.load_gather(table_v, [ds_[g]])
                        out.append(ms[g] < chk)
                    return tuple(out)

                lax.while_loop(cond, body, tuple(lost))

            return carry2

        lax.fori_loop(0, NBATCH, batch_body, 0)
        return carry

    lax.fori_loop(0, NCHUNKS, chunk_body, 0)
    pltpu.sync_copy(table_v, part_hbm.at[pl.ds(wid * NPAD, NPAD)])


@functools.partial(
    pl.kernel,
    mesh=_mesh,
    out_type=jax.ShapeDtypeStruct((NPAD,), jnp.float32),
    compiler_params=_sc_params,
    scratch_types=[
        pltpu.VMEM((NTILES * COLS,), jnp.float32),
        pltpu.VMEM((COLS,), jnp.float32),
        pltpu.SemaphoreType.DMA,
    ],
)
def _merge_kernel(part_hbm, md_hbm, rows_v, acc_v, sem):
    wid = _wid()
    base = wid * COLS
    copies = [
        pltpu.async_copy(part_hbm.at[pl.ds(r * NPAD + base, COLS)],
                         rows_v.at[pl.ds(r * COLS, COLS)], sem)
        for r in range(NTILES)
    ]
    for cp in copies:
        cp.wait()

    def col_body(j, carry):
        a = rows_v[pl.ds(j * 16, 16)]
        for r in range(1, NTILES):
            a = jnp.minimum(a, rows_v[pl.ds(r * COLS + j * 16, 16)])
        acc_v[pl.ds(j * 16, 16)] = a
        return carry

    lax.fori_loop(0, COLS // 16, col_body, 0)
    pltpu.sync_copy(acc_v, md_hbm.at[pl.ds(base, COLS)])


ROWS = 1024
GRID = NPAD // ROWS


def _mlp_body(hg_ref, gi_ref, w1a_ref, w1b_ref, b1_ref, w2_ref, b2_ref,
              temp_ref):
    x = (jnp.dot(hg_ref[...], w1a_ref[...], preferred_element_type=jnp.float32)
         + jnp.dot(gi_ref[...], w1b_ref[...], preferred_element_type=jnp.float32)
         + b1_ref[...])
    h1 = jnp.where(x > 0, x, jnp.exp(x) - 1.0)
    t = jnp.dot(h1, w2_ref[...], preferred_element_type=jnp.float32) + b2_ref[...]
    temp = 1.0 / (1.0 + jnp.exp(-t))
    temp_ref[...] = jnp.transpose(temp).reshape(1, 1, ROWS)


_row_spec = pl.BlockSpec((ROWS, 256), lambda i: (i, 0))
_fix = lambda i: (0, 0)
_lane_spec = pl.BlockSpec((1, 1, ROWS), lambda i: (i, 0, 0))

_mlp_call = pl.pallas_call(
    _mlp_body,
    grid=(GRID,),
    in_specs=[
        _row_spec,
        _row_spec,
        pl.BlockSpec((256, 256), _fix),
        pl.BlockSpec((256, 256), _fix),
        pl.BlockSpec((1, 256), _fix),
        pl.BlockSpec((256, 1), _fix),
        pl.BlockSpec((1, 1), _fix),
    ],
    out_specs=_lane_spec,
    out_shape=jax.ShapeDtypeStruct((GRID, 1, ROWS), jnp.float32),
)


def _epilogue_body(md_ref, temp_ref, beta_ref, out_ref):
    md = jnp.minimum(md_ref[...].reshape(1, ROWS), 1e4)
    temp = temp_ref[...].reshape(1, ROWS)
    dw = beta_ref[...] ** 2
    dscore = jnp.exp((-dw) * md)
    switch = dscore * temp
    hard = (switch >= 0.5).astype(jnp.float32)
    label = (hard - switch) + switch
    out_ref[...] = jnp.concatenate(
        [dscore, temp, switch, label], axis=0).reshape(1, 4, ROWS)


_epilogue_call = pl.pallas_call(
    _epilogue_body,
    grid=(GRID,),
    in_specs=[
        _lane_spec,
        _lane_spec,
        pl.BlockSpec((1, 1), _fix),
    ],
    out_specs=pl.BlockSpec((1, 4, ROWS), lambda i: (i, 0, 0)),
    out_shape=jax.ShapeDtypeStruct((GRID, 4, ROWS), jnp.float32),
)


def kernel(distance_to_center, edge_index, h_gru, gru_inp, beta, W1, b1, W2, b2):
    edge_flat = edge_index.reshape(-1)
    init = jnp.full((NPAD,), 1e4, jnp.float32)
    part = _segmin_kernel(distance_to_center, edge_flat, init)
    md = _merge_kernel(part)
    temp49 = _mlp_call(h_gru, gru_inp, W1[:256], W1[256:],
                       b1.reshape(1, 256), W2, b2.reshape(1, 1))
    out = _epilogue_call(md.reshape(GRID, 1, ROWS), temp49, beta)
    res = []
    for k in range(4):
        res.append(out[:, k, :].reshape(NPAD)[:N])
    return tuple(res)


# speedup vs baseline: 80.2013x; 1.1290x over previous
import functools

import jax
import jax.numpy as jnp
from jax import lax
from jax.experimental import pallas as pl
from jax.experimental.pallas import tpu as pltpu
from jax.experimental.pallas import tpu_sc as plsc

N = 50000
E = 1600000
NTILES = 32
NCORES = 2
COLS = 1568
NPAD = NTILES * COLS
EPT = E // NTILES
CHUNK = 2000
NCHUNKS = EPT // CHUNK
G = 5
BATCH = G * 16
NBATCH = CHUNK // BATCH

_mesh = plsc.VectorSubcoreMesh(core_axis_name="c", subcore_axis_name="s")
_sc_params = pltpu.CompilerParams(needs_layout_passes=False)


def _wid():
    return lax.axis_index("s") * NCORES + lax.axis_index("c")


@functools.partial(
    pl.kernel,
    mesh=_mesh,
    out_type=jax.ShapeDtypeStruct((NTILES * NPAD,), jnp.float32),
    compiler_params=_sc_params,
    scratch_types=[
        pltpu.VMEM((N,), jnp.float32),
        pltpu.VMEM((NPAD,), jnp.float32),
        pltpu.VMEM((2 * CHUNK,), jnp.int32),
        pltpu.VMEM((2 * CHUNK,), jnp.int32),
        pltpu.SemaphoreType.DMA,
    ],
)
def _segmin_kernel(dist_hbm, edge_hbm, init_hbm, part_hbm,
                   dist_v, table_v, src_v, dst_v, sem):
    wid = _wid()
    base_e = wid * EPT
    pltpu.async_copy(edge_hbm.at[pl.ds(base_e, CHUNK)],
                     src_v.at[pl.ds(0, CHUNK)], sem)
    pltpu.async_copy(edge_hbm.at[pl.ds(E + base_e, CHUNK)],
                     dst_v.at[pl.ds(0, CHUNK)], sem)
    pltpu.sync_copy(dist_hbm, dist_v)
    pltpu.sync_copy(init_hbm, table_v)

    def chunk_body(c, carry):
        par = lax.rem(c, 2) * CHUNK
        pltpu.make_async_copy(edge_hbm.at[pl.ds(0, CHUNK)],
                              src_v.at[pl.ds(par, CHUNK)], sem).wait()
        pltpu.make_async_copy(edge_hbm.at[pl.ds(0, CHUNK)],
                              dst_v.at[pl.ds(par, CHUNK)], sem).wait()

        nxt = (CHUNK - par)

        @pl.when(c + 1 < NCHUNKS)
        def _():
            off = base_e + (c + 1) * CHUNK
            pltpu.async_copy(edge_hbm.at[pl.ds(off, CHUNK)],
                             src_v.at[pl.ds(nxt, CHUNK)], sem)
            pltpu.async_copy(edge_hbm.at[pl.ds(E + off, CHUNK)],
                             dst_v.at[pl.ds(nxt, CHUNK)], sem)

        def batch_body(j, carry2):
            b0 = par + j * BATCH
            ss = [src_v[pl.ds(b0 + g * 16, 16)] for g in range(G)]
            ds_ = [dst_v[pl.ds(b0 + g * 16, 16)] for g in range(G)]
            ms = [plsc.load_gather(dist_v, [s]) for s in ss]
            curs = [plsc.load_gather(table_v, [d]) for d in ds_]
            for g in range(G):
                plsc.store_scatter(table_v, [ds_[g]],
                                   jnp.minimum(curs[g], ms[g]))
            cur2 = [plsc.load_gather(table_v, [d]) for d in ds_]
            lost = [ms[g] < cur2[g] for g in range(G)]
            anyl = lost[0]
            for g in range(1, G):
                anyl = anyl | lost[g]

            @pl.when(jnp.any(anyl))
            def _():
                def cond(masks):
                    a = masks[0]
                    for g in range(1, G):
                        a = a | masks[g]
                    return jnp.any(a)

                def body(masks):
                    out = []
                    for g in range(G):
                        cur = plsc.load_gather(table_v, [ds_[g]])
                        plsc.store_scatter(table_v, [ds_[g]],
                                           jnp.minimum(cur, ms[g]),
                                           mask=masks[g])
                        chk = plsc.load_gather(table_v, [ds_[g]])
                        out.append(ms[g] < chk)
                    return tuple(out)

                lax.while_loop(cond, body, tuple(lost))

            return carry2

        lax.fori_loop(0, NBATCH, batch_body, 0)
        return carry

    lax.fori_loop(0, NCHUNKS, chunk_body, 0)
    pltpu.sync_copy(table_v, part_hbm.at[pl.ds(wid * NPAD, NPAD)])


@functools.partial(
    pl.kernel,
    mesh=_mesh,
    out_type=jax.ShapeDtypeStruct((NPAD,), jnp.float32),
    compiler_params=_sc_params,
    scratch_types=[
        pltpu.VMEM((NTILES * COLS,), jnp.float32),
        pltpu.VMEM((COLS,), jnp.float32),
        pltpu.SemaphoreType.DMA,
    ],
)
def _merge_kernel(part_hbm, md_hbm, rows_v, acc_v, sem):
    wid = _wid()
    base = wid * COLS
    copies = [
        pltpu.async_copy(part_hbm.at[pl.ds(r * NPAD + base, COLS)],
                         rows_v.at[pl.ds(r * COLS, COLS)], sem)
        for r in range(NTILES)
    ]
    for cp in copies:
        cp.wait()

    def col_body(j, carry):
        a = rows_v[pl.ds(j * 16, 16)]
        for r in range(1, NTILES):
            a = jnp.minimum(a, rows_v[pl.ds(r * COLS + j * 16, 16)])
        acc_v[pl.ds(j * 16, 16)] = a
        return carry

    lax.fori_loop(0, COLS // 16, col_body, 0)
    pltpu.sync_copy(acc_v, md_hbm.at[pl.ds(base, COLS)])


ROWS = 1024
GRID = NPAD // ROWS


def _mlp_body(hg_ref, gi_ref, w1a_ref, w1b_ref, b1_ref, w2_ref, b2_ref,
              temp_ref):
    x = (jnp.dot(hg_ref[...], w1a_ref[...], preferred_element_type=jnp.float32)
         + jnp.dot(gi_ref[...], w1b_ref[...], preferred_element_type=jnp.float32)
         + b1_ref[...])
    h1 = jnp.where(x > 0, x, jnp.exp(x) - 1.0)
    t = jnp.dot(h1, w2_ref[...], preferred_element_type=jnp.float32) + b2_ref[...]
    temp = 1.0 / (1.0 + jnp.exp(-t))
    temp_ref[...] = jnp.transpose(temp).reshape(1, 1, ROWS)


_row_spec = pl.BlockSpec((ROWS, 256), lambda i: (i, 0))
_fix = lambda i: (0, 0)
_lane_spec = pl.BlockSpec((1, 1, ROWS), lambda i: (i, 0, 0))

_mlp_call = pl.pallas_call(
    _mlp_body,
    grid=(GRID,),
    in_specs=[
        _row_spec,
        _row_spec,
        pl.BlockSpec((256, 256), _fix),
        pl.BlockSpec((256, 256), _fix),
        pl.BlockSpec((1, 256), _fix),
        pl.BlockSpec((256, 1), _fix),
        pl.BlockSpec((1, 1), _fix),
    ],
    out_specs=_lane_spec,
    out_shape=jax.ShapeDtypeStruct((GRID, 1, ROWS), jnp.float32),
)


def _epilogue_body(md_ref, temp_ref, beta_ref, out_ref):
    md = jnp.minimum(md_ref[...], 1e4)
    temp = temp_ref[...]
    dw = beta_ref[0, 0] ** 2
    dscore = jnp.exp((-dw) * md)
    switch = dscore * temp
    hard = (switch >= 0.5).astype(jnp.float32)
    label = (hard - switch) + switch
    out_ref[...] = jnp.concatenate(
        [dscore, temp, switch, label], axis=1)


_epilogue_call = pl.pallas_call(
    _epilogue_body,
    out_shape=jax.ShapeDtypeStruct((GRID, 4, ROWS), jnp.float32),
)


def kernel(distance_to_center, edge_index, h_gru, gru_inp, beta, W1, b1, W2, b2):
    edge_flat = edge_index.reshape(-1)
    init = jnp.full((NPAD,), 1e4, jnp.float32)
    part = _segmin_kernel(distance_to_center, edge_flat, init)
    temp49 = _mlp_call(h_gru, gru_inp, W1[:256], W1[256:],
                       b1.reshape(1, 256), W2, b2.reshape(1, 1))
    md = _merge_kernel(part)
    out = _epilogue_call(md.reshape(GRID, 1, ROWS), temp49, beta)
    res = []
    for k in range(4):
        res.append(out[:, k, :].reshape(NPAD)[:N])
    return tuple(res)


# speedup vs baseline: 99.5647x; 1.2414x over previous
import functools

import jax
import jax.numpy as jnp
from jax import lax
from jax.experimental import pallas as pl
from jax.experimental.pallas import tpu as pltpu
from jax.experimental.pallas import tpu_sc as plsc

N = 50000
E = 1600000
NTILES = 32
NCORES = 2
COLS = 1568
NPAD = NTILES * COLS
EPT = E // NTILES
CHUNK = 2000
NCHUNKS = EPT // CHUNK
G = 5
BATCH = G * 16
NBATCH = CHUNK // BATCH

_mesh = plsc.VectorSubcoreMesh(core_axis_name="c", subcore_axis_name="s")
_sc_params = pltpu.CompilerParams(needs_layout_passes=False,
                                  use_tc_tiling_on_sc=False)


def _wid():
    return lax.axis_index("s") * NCORES + lax.axis_index("c")


@functools.partial(
    pl.kernel,
    mesh=_mesh,
    out_type=jax.ShapeDtypeStruct((NTILES * NPAD,), jnp.float32),
    compiler_params=_sc_params,
    scratch_types=[
        pltpu.VMEM((N,), jnp.float32),
        pltpu.VMEM((NPAD,), jnp.float32),
        pltpu.VMEM((2 * CHUNK,), jnp.int32),
        pltpu.VMEM((2 * CHUNK,), jnp.int32),
        pltpu.SemaphoreType.DMA,
    ],
)
def _segmin_kernel(dist_hbm, edge_hbm, init_hbm, part_hbm,
                   dist_v, table_v, src_v, dst_v, sem):
    wid = _wid()
    base_e = wid * EPT
    pltpu.async_copy(edge_hbm.at[0, pl.ds(base_e, CHUNK)],
                     src_v.at[pl.ds(0, CHUNK)], sem)
    pltpu.async_copy(edge_hbm.at[1, pl.ds(base_e, CHUNK)],
                     dst_v.at[pl.ds(0, CHUNK)], sem)
    pltpu.sync_copy(dist_hbm, dist_v)
    pltpu.sync_copy(init_hbm, table_v)

    def chunk_body(c, carry):
        par = lax.rem(c, 2) * CHUNK
        pltpu.make_async_copy(edge_hbm.at[0, pl.ds(0, CHUNK)],
                              src_v.at[pl.ds(par, CHUNK)], sem).wait()
        pltpu.make_async_copy(edge_hbm.at[0, pl.ds(0, CHUNK)],
                              dst_v.at[pl.ds(par, CHUNK)], sem).wait()

        nxt = (CHUNK - par)

        @pl.when(c + 1 < NCHUNKS)
        def _():
            off = base_e + (c + 1) * CHUNK
            pltpu.async_copy(edge_hbm.at[0, pl.ds(off, CHUNK)],
                             src_v.at[pl.ds(nxt, CHUNK)], sem)
            pltpu.async_copy(edge_hbm.at[1, pl.ds(off, CHUNK)],
                             dst_v.at[pl.ds(nxt, CHUNK)], sem)

        def batch_body(j, carry2):
            b0 = par + j * BATCH
            ss = [src_v[pl.ds(b0 + g * 16, 16)] for g in range(G)]
            ds_ = [dst_v[pl.ds(b0 + g * 16, 16)] for g in range(G)]
            ms = [plsc.load_gather(dist_v, [s]) for s in ss]
            curs = [plsc.load_gather(table_v, [d]) for d in ds_]
            for g in range(G):
                plsc.store_scatter(table_v, [ds_[g]],
                                   jnp.minimum(curs[g], ms[g]))
            cur2 = [plsc.load_gather(table_v, [d]) for d in ds_]
            lost = [ms[g] < cur2[g] for g in range(G)]
            anyl = lost[0]
            for g in range(1, G):
                anyl = anyl | lost[g]

            @pl.when(jnp.any(anyl))
            def _():
                def cond(masks):
                    a = masks[0]
                    for g in range(1, G):
                        a = a | masks[g]
                    return jnp.any(a)

                def body(masks):
                    out = []
                    for g in range(G):
                        cur = plsc.load_gather(table_v, [ds_[g]])
                        plsc.store_scatter(table_v, [ds_[g]],
                                           jnp.minimum(cur, ms[g]),
                                           mask=masks[g])
                        chk = plsc.load_gather(table_v, [ds_[g]])
                        out.append(ms[g] < chk)
                    return tuple(out)

                lax.while_loop(cond, body, tuple(lost))

            return carry2

        lax.fori_loop(0, NBATCH, batch_body, 0)
        return carry

    lax.fori_loop(0, NCHUNKS, chunk_body, 0)
    pltpu.sync_copy(table_v, part_hbm.at[pl.ds(wid * NPAD, NPAD)])


@functools.partial(
    pl.kernel,
    mesh=_mesh,
    out_type=jax.ShapeDtypeStruct((NPAD,), jnp.float32),
    compiler_params=_sc_params,
    scratch_types=[
        pltpu.VMEM((NTILES * COLS,), jnp.float32),
        pltpu.VMEM((COLS,), jnp.float32),
        pltpu.SemaphoreType.DMA,
    ],
)
def _merge_kernel(part_hbm, temp_hbm, md_hbm, rows_v, acc_v, sem):
    del temp_hbm
    wid = _wid()
    base = wid * COLS
    copies = [
        pltpu.async_copy(part_hbm.at[pl.ds(r * NPAD + base, COLS)],
                         rows_v.at[pl.ds(r * COLS, COLS)], sem)
        for r in range(NTILES)
    ]
    for cp in copies:
        cp.wait()

    def col_body(j, carry):
        a = rows_v[pl.ds(j * 16, 16)]
        for r in range(1, NTILES):
            a = jnp.minimum(a, rows_v[pl.ds(r * COLS + j * 16, 16)])
        acc_v[pl.ds(j * 16, 16)] = a
        return carry

    lax.fori_loop(0, COLS // 16, col_body, 0)
    pltpu.sync_copy(acc_v, md_hbm.at[pl.ds(base, COLS)])


ROWS = 1024
GRID = NPAD // ROWS


def _mlp_body(hg_ref, gi_ref, w1a_ref, w1b_ref, b1_ref, w2_ref, b2_ref,
              temp_ref):
    x = (jnp.dot(hg_ref[...], w1a_ref[...], preferred_element_type=jnp.float32)
         + jnp.dot(gi_ref[...], w1b_ref[...], preferred_element_type=jnp.float32)
         + b1_ref[...])
    h1 = jnp.where(x > 0, x, jnp.exp(x) - 1.0)
    t = jnp.dot(h1, w2_ref[...], preferred_element_type=jnp.float32) + b2_ref[...]
    temp = 1.0 / (1.0 + jnp.exp(-t))
    temp_ref[...] = jnp.transpose(temp).reshape(1, 1, ROWS)


_row_spec = pl.BlockSpec((ROWS, 256), lambda i: (i, 0))
_fix = lambda i: (0, 0)
_lane_spec = pl.BlockSpec((1, 1, ROWS), lambda i: (i, 0, 0))

_mlp_call = pl.pallas_call(
    _mlp_body,
    grid=(GRID,),
    in_specs=[
        _row_spec,
        _row_spec,
        pl.BlockSpec((256, 256), _fix),
        pl.BlockSpec((256, 256), _fix),
        pl.BlockSpec((1, 256), _fix),
        pl.BlockSpec((256, 1), _fix),
        pl.BlockSpec((1, 1), _fix),
    ],
    out_specs=_lane_spec,
    out_shape=jax.ShapeDtypeStruct((GRID, 1, ROWS), jnp.float32),
)


def _epilogue_body(md_ref, temp_ref, beta_ref, out_ref):
    md = jnp.minimum(md_ref[...], 1e4)
    temp = temp_ref[...]
    dw = beta_ref[0, 0] ** 2
    dscore = jnp.exp((-dw) * md)
    switch = dscore * temp
    hard = (switch >= 0.5).astype(jnp.float32)
    label = (hard - switch) + switch
    out_ref[...] = jnp.concatenate(
        [dscore, temp, switch, label], axis=1)


_epilogue_call = pl.pallas_call(
    _epilogue_body,
    out_shape=jax.ShapeDtypeStruct((GRID, 4, ROWS), jnp.float32),
)


def kernel(distance_to_center, edge_index, h_gru, gru_inp, beta, W1, b1, W2, b2):
    init = jnp.full((NPAD,), 1e4, jnp.float32)
    part = _segmin_kernel(distance_to_center, edge_index, init)
    temp49 = _mlp_call(h_gru, gru_inp, W1[:256], W1[256:],
                       b1.reshape(1, 256), W2, b2.reshape(1, 1))
    md = _merge_kernel(part, temp49)
    out = _epilogue_call(md.reshape(GRID, 1, ROWS), temp49, beta)
    res = []
    for k in range(4):
        res.append(out[:, k, :].reshape(NPAD)[:N])
    return tuple(res)


# speedup vs baseline: 104.2980x; 1.0475x over previous
import functools

import jax
import jax.numpy as jnp
from jax import lax
from jax.experimental import pallas as pl
from jax.experimental.pallas import tpu as pltpu
from jax.experimental.pallas import tpu_sc as plsc

N = 50000
E = 1600000
NTILES = 32
NCORES = 2
COLS = 1568
NPAD = NTILES * COLS
EPT = E // NTILES
CHUNK = 2000
NCHUNKS = EPT // CHUNK
G = 5
BATCH = G * 16
NBATCH = CHUNK // BATCH

_mesh = plsc.VectorSubcoreMesh(core_axis_name="c", subcore_axis_name="s")
_sc_params = pltpu.CompilerParams(needs_layout_passes=False,
                                  use_tc_tiling_on_sc=False)


def _wid():
    return lax.axis_index("s") * NCORES + lax.axis_index("c")


@functools.partial(
    pl.kernel,
    mesh=_mesh,
    out_type=jax.ShapeDtypeStruct((NTILES * NPAD,), jnp.float32),
    compiler_params=_sc_params,
    scratch_types=[
        pltpu.VMEM((N,), jnp.float32),
        pltpu.VMEM((NPAD,), jnp.float32),
        pltpu.VMEM((2 * CHUNK,), jnp.int32),
        pltpu.VMEM((2 * CHUNK,), jnp.int32),
        pltpu.SemaphoreType.DMA,
    ],
)
def _segmin_kernel(dist_hbm, edge_hbm, init_hbm, part_hbm,
                   dist_v, table_v, src_v, dst_v, sem):
    wid = _wid()
    base_e = wid * EPT
    pltpu.async_copy(edge_hbm.at[0, pl.ds(base_e, CHUNK)],
                     src_v.at[pl.ds(0, CHUNK)], sem)
    pltpu.async_copy(edge_hbm.at[1, pl.ds(base_e, CHUNK)],
                     dst_v.at[pl.ds(0, CHUNK)], sem)
    pltpu.sync_copy(dist_hbm, dist_v)
    pltpu.sync_copy(init_hbm, table_v)

    def chunk_body(c, carry):
        par = lax.rem(c, 2) * CHUNK
        pltpu.make_async_copy(edge_hbm.at[0, pl.ds(0, CHUNK)],
                              src_v.at[pl.ds(par, CHUNK)], sem).wait()
        pltpu.make_async_copy(edge_hbm.at[0, pl.ds(0, CHUNK)],
                              dst_v.at[pl.ds(par, CHUNK)], sem).wait()

        nxt = (CHUNK - par)

        @pl.when(c + 1 < NCHUNKS)
        def _():
            off = base_e + (c + 1) * CHUNK
            pltpu.async_copy(edge_hbm.at[0, pl.ds(off, CHUNK)],
                             src_v.at[pl.ds(nxt, CHUNK)], sem)
            pltpu.async_copy(edge_hbm.at[1, pl.ds(off, CHUNK)],
                             dst_v.at[pl.ds(nxt, CHUNK)], sem)

        def batch_body(j, carry2):
            b0 = par + j * BATCH
            ss = [src_v[pl.ds(b0 + g * 16, 16)] for g in range(G)]
            ds_ = [dst_v[pl.ds(b0 + g * 16, 16)] for g in range(G)]
            ms = [plsc.load_gather(dist_v, [s]) for s in ss]
            curs = [plsc.load_gather(table_v, [d]) for d in ds_]
            for g in range(G):
                plsc.store_scatter(table_v, [ds_[g]],
                                   jnp.minimum(curs[g], ms[g]))
            cur2 = [plsc.load_gather(table_v, [d]) for d in ds_]
            lost = [ms[g] < cur2[g] for g in range(G)]
            anyl = lost[0]
            for g in range(1, G):
                anyl = anyl | lost[g]

            @pl.when(jnp.any(anyl))
            def _():
                def cond(masks):
                    a = masks[0]
                    for g in range(1, G):
                        a = a | masks[g]
                    return jnp.any(a)

                def body(masks):
                    out = []
                    for g in range(G):
                        cur = plsc.load_gather(table_v, [ds_[g]])
                        plsc.store_scatter(table_v, [ds_[g]],
                                           jnp.minimum(cur, ms[g]),
                                           mask=masks[g])
                        chk = plsc.load_gather(table_v, [ds_[g]])
                        out.append(ms[g] < chk)
                    return tuple(out)

                lax.while_loop(cond, body, tuple(lost))

            return carry2

        lax.fori_loop(0, NBATCH, batch_body, 0)
        return carry

    lax.fori_loop(0, NCHUNKS, chunk_body, 0)
    pltpu.sync_copy(table_v, part_hbm.at[pl.ds(wid * NPAD, NPAD)])


@functools.partial(
    pl.kernel,
    mesh=_mesh,
    out_type=[jax.ShapeDtypeStruct((NPAD,), jnp.float32)] * 3,
    compiler_params=_sc_params,
    scratch_types=[
        pltpu.VMEM((NTILES * COLS,), jnp.float32),
        pltpu.VMEM((COLS,), jnp.float32),
        pltpu.VMEM((COLS,), jnp.float32),
        pltpu.VMEM((COLS,), jnp.float32),
        pltpu.VMEM((COLS,), jnp.float32),
        pltpu.VMEM((16,), jnp.float32),
        pltpu.SemaphoreType.DMA,
    ],
)
def _merge_kernel(part_hbm, temp_hbm, ndw_hbm, ds_hbm, sw_hbm, lb_hbm,
                  rows_v, temp_v, ds_v, sw_v, lb_v, ndw_v, sem):
    wid = _wid()
    base = wid * COLS
    copies = [
        pltpu.async_copy(part_hbm.at[pl.ds(r * NPAD + base, COLS)],
                         rows_v.at[pl.ds(r * COLS, COLS)], sem)
        for r in range(NTILES)
    ]
    copies.append(pltpu.async_copy(temp_hbm.at[pl.ds(base, COLS)], temp_v, sem))
    copies.append(pltpu.async_copy(ndw_hbm, ndw_v, sem))
    for cp in copies:
        cp.wait()
    ndw = ndw_v[...]

    def col_body(j, carry):
        sl = pl.ds(j * 16, 16)
        a = rows_v[sl]
        for r in range(1, NTILES):
            a = jnp.minimum(a, rows_v[pl.ds(r * COLS + j * 16, 16)])
        md = jnp.minimum(a, 1e4)
        dscore = jnp.exp(ndw * md)
        switch = dscore * temp_v[sl]
        hard = jnp.where(switch >= 0.5, 1.0, 0.0)
        ds_v[sl] = dscore
        sw_v[sl] = switch
        lb_v[sl] = (hard - switch) + switch
        return carry

    lax.fori_loop(0, COLS // 16, col_body, 0)
    pltpu.sync_copy(ds_v, ds_hbm.at[pl.ds(base, COLS)])
    pltpu.sync_copy(sw_v, sw_hbm.at[pl.ds(base, COLS)])
    pltpu.sync_copy(lb_v, lb_hbm.at[pl.ds(base, COLS)])


ROWS = 1024
GRID = NPAD // ROWS


def _mlp_body(hg_ref, gi_ref, w1_ref, b1_ref, w2_ref, b2_ref, temp_ref):
    h = jnp.concatenate([hg_ref[...], gi_ref[...]], axis=1)
    x = jnp.dot(h, w1_ref[...], preferred_element_type=jnp.float32) + b1_ref[...]
    h1 = jnp.where(x > 0, x, jnp.exp(x) - 1.0)
    t = jnp.dot(h1, w2_ref[...], preferred_element_type=jnp.float32) + b2_ref[...]
    temp = 1.0 / (1.0 + jnp.exp(-t))
    temp_ref[...] = jnp.transpose(temp).reshape(1, 1, ROWS)


_row_spec = pl.BlockSpec((ROWS, 256), lambda i: (i, 0))
_fix = lambda i: (0, 0)
_lane_spec = pl.BlockSpec((1, 1, ROWS), lambda i: (i, 0, 0))

_mlp_call = pl.pallas_call(
    _mlp_body,
    grid=(GRID,),
    in_specs=[
        _row_spec,
        _row_spec,
        pl.BlockSpec((512, 256), _fix),
        pl.BlockSpec((1, 256), _fix),
        pl.BlockSpec((256, 1), _fix),
        pl.BlockSpec((1, 1), _fix),
    ],
    out_specs=_lane_spec,
    out_shape=jax.ShapeDtypeStruct((GRID, 1, ROWS), jnp.float32),
)


def kernel(distance_to_center, edge_index, h_gru, gru_inp, beta, W1, b1, W2, b2):
    init = jnp.full((NPAD,), 1e4, jnp.float32)
    part = _segmin_kernel(distance_to_center, edge_index, init)
    temp49 = _mlp_call(h_gru, gru_inp, W1, b1.reshape(1, 256), W2,
                       b2.reshape(1, 1))
    temp_flat = temp49.reshape(NPAD)
    ndw = jnp.broadcast_to((-(beta ** 2)).reshape(1), (16,))
    dist_s, switch_s, label_s = _merge_kernel(part, temp_flat, ndw)
    return (dist_s[:N], temp_flat[:N], switch_s[:N], label_s[:N])


# speedup vs baseline: 134.6647x; 1.2912x over previous
import functools

import jax
import jax.numpy as jnp
from jax import lax
from jax.experimental import pallas as pl
from jax.experimental.pallas import tpu as pltpu
from jax.experimental.pallas import tpu_sc as plsc

N = 50000
E = 1600000
NTILES = 32
NCORES = 2
COLS = 1568
NPAD = NTILES * COLS
NROWS = E // 128
ROWS_PER = NROWS // NTILES
EXTRA = NROWS - ROWS_PER * NTILES
CH = 39
NCH = ROWS_PER // CH
G = 8

_mesh = plsc.VectorSubcoreMesh(core_axis_name="c", subcore_axis_name="s")
_sc_params = pltpu.CompilerParams(needs_layout_passes=False,
                                  use_tc_tiling_on_sc=False)


def _wid():
    return lax.axis_index("s") * NCORES + lax.axis_index("c")


@functools.partial(
    pl.kernel,
    mesh=_mesh,
    out_type=jax.ShapeDtypeStruct((NTILES * NPAD,), jnp.float32),
    compiler_params=_sc_params,
    scratch_types=[
        pltpu.VMEM((N,), jnp.float32),
        pltpu.VMEM((NPAD,), jnp.float32),
        pltpu.VMEM((2 * CH, 2, 128), jnp.int32),
        pltpu.SemaphoreType.DMA,
    ],
)
def _segmin_kernel(dist_hbm, edge_hbm, init_hbm, part_hbm,
                   dist_v, table_v, buf_v, sem):
    wid = _wid()
    base_row = wid * ROWS_PER + jnp.minimum(wid, EXTRA)
    pltpu.async_copy(edge_hbm.at[pl.ds(base_row, CH), :, :],
                     buf_v.at[pl.ds(0, CH), :, :], sem)
    pltpu.sync_copy(dist_hbm, dist_v)
    pltpu.sync_copy(init_hbm, table_v)

    def process_row(row):
        ss = [buf_v[row, 0, pl.ds(g * 16, 16)] for g in range(G)]
        ds_ = [buf_v[row, 1, pl.ds(g * 16, 16)] for g in range(G)]
        ms = [plsc.load_gather(dist_v, [s]) for s in ss]
        curs = [plsc.load_gather(table_v, [d]) for d in ds_]
        for g in range(G):
            plsc.store_scatter(table_v, [ds_[g]],
                               jnp.minimum(curs[g], ms[g]))
        cur2 = [plsc.load_gather(table_v, [d]) for d in ds_]
        lost = [ms[g] < cur2[g] for g in range(G)]
        anyl = lost[0]
        for g in range(1, G):
            anyl = anyl | lost[g]

        @pl.when(jnp.any(anyl))
        def _():
            def cond(masks):
                a = masks[0]
                for g in range(1, G):
                    a = a | masks[g]
                return jnp.any(a)

            def body(masks):
                out = []
                for g in range(G):
                    cur = plsc.load_gather(table_v, [ds_[g]])
                    plsc.store_scatter(table_v, [ds_[g]],
                                       jnp.minimum(cur, ms[g]),
                                       mask=masks[g])
                    chk = plsc.load_gather(table_v, [ds_[g]])
                    out.append(ms[g] < chk)
                return tuple(out)

            lax.while_loop(cond, body, tuple(lost))

    def chunk_body(c, carry):
        par = lax.rem(c, 2) * CH
        pltpu.make_async_copy(edge_hbm.at[pl.ds(0, CH), :, :],
                              buf_v.at[pl.ds(par, CH), :, :], sem).wait()

        nxt = CH - par

        @pl.when(c + 1 < NCH)
        def _():
            row0 = base_row + (c + 1) * CH
            pltpu.async_copy(edge_hbm.at[pl.ds(row0, CH), :, :],
                             buf_v.at[pl.ds(nxt, CH), :, :], sem)

        def row_body(j, carry2):
            process_row(par + j)
            return carry2

        lax.fori_loop(0, CH, row_body, 0)
        return carry

    lax.fori_loop(0, NCH, chunk_body, 0)

    @pl.when(wid < EXTRA)
    def _():
        pltpu.sync_copy(edge_hbm.at[pl.ds(base_row + ROWS_PER, 1), :, :],
                        buf_v.at[pl.ds(0, 1), :, :])
        process_row(0)

    pltpu.sync_copy(table_v, part_hbm.at[pl.ds(wid * NPAD, NPAD)])


@functools.partial(
    pl.kernel,
    mesh=_mesh,
    out_type=[jax.ShapeDtypeStruct((NPAD,), jnp.float32)] * 3,
    compiler_params=_sc_params,
    scratch_types=[
        pltpu.VMEM((NTILES * COLS,), jnp.float32),
        pltpu.VMEM((COLS,), jnp.float32),
        pltpu.VMEM((COLS,), jnp.float32),
        pltpu.VMEM((COLS,), jnp.float32),
        pltpu.VMEM((COLS,), jnp.float32),
        pltpu.VMEM((16,), jnp.float32),
        pltpu.SemaphoreType.DMA,
    ],
)
def _merge_kernel(part_hbm, temp_hbm, ndw_hbm, ds_hbm, sw_hbm, lb_hbm,
                  rows_v, temp_v, ds_v, sw_v, lb_v, ndw_v, sem):
    wid = _wid()
    base = wid * COLS
    copies = [
        pltpu.async_copy(part_hbm.at[pl.ds(r * NPAD + base, COLS)],
                         rows_v.at[pl.ds(r * COLS, COLS)], sem)
        for r in range(NTILES)
    ]
    copies.append(pltpu.async_copy(temp_hbm.at[pl.ds(base, COLS)], temp_v, sem))
    copies.append(pltpu.async_copy(ndw_hbm, ndw_v, sem))
    for cp in copies:
        cp.wait()
    ndw = ndw_v[...]

    def col_body(j, carry):
        sl = pl.ds(j * 16, 16)
        a = rows_v[sl]
        for r in range(1, NTILES):
            a = jnp.minimum(a, rows_v[pl.ds(r * COLS + j * 16, 16)])
        md = jnp.minimum(a, 1e4)
        dscore = jnp.exp(ndw * md)
        switch = dscore * temp_v[sl]
        hard = jnp.where(switch >= 0.5, 1.0, 0.0)
        ds_v[sl] = dscore
        sw_v[sl] = switch
        lb_v[sl] = (hard - switch) + switch
        return carry

    lax.fori_loop(0, COLS // 16, col_body, 0)
    pltpu.sync_copy(ds_v, ds_hbm.at[pl.ds(base, COLS)])
    pltpu.sync_copy(sw_v, sw_hbm.at[pl.ds(base, COLS)])
    pltpu.sync_copy(lb_v, lb_hbm.at[pl.ds(base, COLS)])


ROWS = 1024
GRID = NPAD // ROWS


def _mlp_body(hg_ref, gi_ref, w1_ref, b1_ref, w2_ref, b2_ref, temp_ref):
    h = jnp.concatenate([hg_ref[...], gi_ref[...]], axis=1)
    x = jnp.dot(h, w1_ref[...], preferred_element_type=jnp.float32) + b1_ref[...]
    h1 = jnp.where(x > 0, x, jnp.exp(x) - 1.0)
    t = jnp.dot(h1, w2_ref[...], preferred_element_type=jnp.float32) + b2_ref[...]
    temp = 1.0 / (1.0 + jnp.exp(-t))
    temp_ref[...] = jnp.transpose(temp).reshape(1, 1, ROWS)


_row_spec = pl.BlockSpec((ROWS, 256), lambda i: (i, 0))
_fix = lambda i: (0, 0)
_lane_spec = pl.BlockSpec((1, 1, ROWS), lambda i: (i, 0, 0))

_mlp_call = pl.pallas_call(
    _mlp_body,
    grid=(GRID,),
    in_specs=[
        _row_spec,
        _row_spec,
        pl.BlockSpec((512, 256), _fix),
        pl.BlockSpec((1, 256), _fix),
        pl.BlockSpec((256, 1), _fix),
        pl.BlockSpec((1, 1), _fix),
    ],
    out_specs=_lane_spec,
    out_shape=jax.ShapeDtypeStruct((GRID, 1, ROWS), jnp.float32),
)


def kernel(distance_to_center, edge_index, h_gru, gru_inp, beta, W1, b1, W2, b2):
    edge3 = edge_index.reshape(2, NROWS, 128).transpose(1, 0, 2)
    init = jnp.full((NPAD,), 1e4, jnp.float32)
    part = _segmin_kernel(distance_to_center, edge3, init)
    temp49 = _mlp_call(h_gru, gru_inp, W1, b1.reshape(1, 256), W2,
                       b2.reshape(1, 1))
    temp_flat = temp49.reshape(NPAD)
    ndw = jnp.broadcast_to((-(beta ** 2)).reshape(1), (16,))
    dist_s, switch_s, label_s = _merge_kernel(part, temp_flat, ndw)
    return (dist_s[:N], temp_flat[:N], switch_s[:N], label_s[:N])


# speedup vs baseline: 140.7035x; 1.0448x over previous
import functools

import jax
import jax.numpy as jnp
from jax import lax
from jax.experimental import pallas as pl
from jax.experimental.pallas import tpu as pltpu
from jax.experimental.pallas import tpu_sc as plsc

N = 50000
E = 1600000
NTILES = 32
NCORES = 2
COLS = 1568
NPAD = NTILES * COLS
NROWS = E // 128
ROWS_PER = NROWS // NTILES
EXTRA = NROWS - ROWS_PER * NTILES
CH = 39
NCH = ROWS_PER // CH
G = 8

_mesh = plsc.VectorSubcoreMesh(core_axis_name="c", subcore_axis_name="s")
_sc_params = pltpu.CompilerParams(needs_layout_passes=False,
                                  use_tc_tiling_on_sc=False)


def _wid():
    return lax.axis_index("s") * NCORES + lax.axis_index("c")


@functools.partial(
    pl.kernel,
    mesh=_mesh,
    out_type=jax.ShapeDtypeStruct((NTILES * NPAD,), jnp.float32),
    compiler_params=_sc_params,
    scratch_types=[
        pltpu.VMEM((N,), jnp.float32),
        pltpu.VMEM((NPAD,), jnp.float32),
        pltpu.VMEM((2 * CH, 2, 128), jnp.int32),
        pltpu.SemaphoreType.DMA,
    ],
)
def _segmin_kernel(dist_hbm, edge_hbm, part_hbm,
                   dist_v, table_v, buf_v, sem):
    wid = _wid()
    base_row = wid * ROWS_PER + jnp.minimum(wid, EXTRA)
    pltpu.async_copy(edge_hbm.at[pl.ds(base_row, CH), :, :],
                     buf_v.at[pl.ds(0, CH), :, :], sem)
    pltpu.sync_copy(dist_hbm, dist_v)
    fill = jnp.full((16,), 1e4, jnp.float32)

    def init_body(i, carry):
        table_v[pl.ds(i * 64, 16)] = fill
        table_v[pl.ds(i * 64 + 16, 16)] = fill
        table_v[pl.ds(i * 64 + 32, 16)] = fill
        table_v[pl.ds(i * 64 + 48, 16)] = fill
        return carry

    lax.fori_loop(0, NPAD // 64, init_body, 0)

    def process_row(row):
        ss = [buf_v[row, 0, pl.ds(g * 16, 16)] for g in range(G)]
        ds_ = [buf_v[row, 1, pl.ds(g * 16, 16)] for g in range(G)]
        ms = [plsc.load_gather(dist_v, [s]) for s in ss]
        curs = [plsc.load_gather(table_v, [d]) for d in ds_]
        for g in range(G):
            plsc.store_scatter(table_v, [ds_[g]],
                               jnp.minimum(curs[g], ms[g]))
        cur2 = [plsc.load_gather(table_v, [d]) for d in ds_]
        lost = [ms[g] < cur2[g] for g in range(G)]
        anyl = lost[0]
        for g in range(1, G):
            anyl = anyl | lost[g]

        @pl.when(jnp.any(anyl))
        def _():
            def cond(masks):
                a = masks[0]
                for g in range(1, G):
                    a = a | masks[g]
                return jnp.any(a)

            def body(masks):
                out = []
                for g in range(G):
                    cur = plsc.load_gather(table_v, [ds_[g]])
                    plsc.store_scatter(table_v, [ds_[g]],
                                       jnp.minimum(cur, ms[g]),
                                       mask=masks[g])
                    chk = plsc.load_gather(table_v, [ds_[g]])
                    out.append(ms[g] < chk)
                return tuple(out)

            lax.while_loop(cond, body, tuple(lost))

    def chunk_body(c, carry):
        par = lax.rem(c, 2) * CH
        pltpu.make_async_copy(edge_hbm.at[pl.ds(0, CH), :, :],
                              buf_v.at[pl.ds(par, CH), :, :], sem).wait()

        nxt = CH - par

        @pl.when(c + 1 < NCH)
        def _():
            row0 = base_row + (c + 1) * CH
            pltpu.async_copy(edge_hbm.at[pl.ds(row0, CH), :, :],
                             buf_v.at[pl.ds(nxt, CH), :, :], sem)

        def row_body(j, carry2):
            process_row(par + j)
            return carry2

        lax.fori_loop(0, CH, row_body, 0)
        return carry

    lax.fori_loop(0, NCH, chunk_body, 0)

    @pl.when(wid < EXTRA)
    def _():
        pltpu.sync_copy(edge_hbm.at[pl.ds(base_row + ROWS_PER, 1), :, :],
                        buf_v.at[pl.ds(0, 1), :, :])
        process_row(0)

    pltpu.sync_copy(table_v, part_hbm.at[pl.ds(wid * NPAD, NPAD)])


@functools.partial(
    pl.kernel,
    mesh=_mesh,
    out_type=[jax.ShapeDtypeStruct((NPAD,), jnp.float32)] * 3,
    compiler_params=_sc_params,
    scratch_types=[
        pltpu.VMEM((NTILES * COLS,), jnp.float32),
        pltpu.VMEM((COLS,), jnp.float32),
        pltpu.VMEM((COLS,), jnp.float32),
        pltpu.VMEM((COLS,), jnp.float32),
        pltpu.VMEM((COLS,), jnp.float32),
        pltpu.VMEM((16,), jnp.float32),
        pltpu.SemaphoreType.DMA,
    ],
)
def _merge_kernel(part_hbm, temp_hbm, ndw_hbm, ds_hbm, sw_hbm, lb_hbm,
                  rows_v, temp_v, ds_v, sw_v, lb_v, ndw_v, sem):
    wid = _wid()
    base = wid * COLS
    copies = [
        pltpu.async_copy(part_hbm.at[pl.ds(r * NPAD + base, COLS)],
                         rows_v.at[pl.ds(r * COLS, COLS)], sem)
        for r in range(NTILES)
    ]
    copies.append(pltpu.async_copy(temp_hbm.at[pl.ds(base, COLS)], temp_v, sem))
    copies.append(pltpu.async_copy(ndw_hbm, ndw_v, sem))
    for cp in copies:
        cp.wait()
    ndw = ndw_v[...]

    def col_body(j, carry):
        sl = pl.ds(j * 16, 16)
        a = rows_v[sl]
        for r in range(1, NTILES):
            a = jnp.minimum(a, rows_v[pl.ds(r * COLS + j * 16, 16)])
        md = jnp.minimum(a, 1e4)
        dscore = jnp.exp(ndw * md)
        switch = dscore * temp_v[sl]
        hard = jnp.where(switch >= 0.5, 1.0, 0.0)
        ds_v[sl] = dscore
        sw_v[sl] = switch
        lb_v[sl] = (hard - switch) + switch
        return carry

    lax.fori_loop(0, COLS // 16, col_body, 0)
    pltpu.sync_copy(ds_v, ds_hbm.at[pl.ds(base, COLS)])
    pltpu.sync_copy(sw_v, sw_hbm.at[pl.ds(base, COLS)])
    pltpu.sync_copy(lb_v, lb_hbm.at[pl.ds(base, COLS)])


ROWS = 1024
GRID = NPAD // ROWS


def _mlp_body(hg_ref, gi_ref, w1_ref, b1_ref, w2_ref, b2_ref, temp_ref):
    h = jnp.concatenate([hg_ref[...], gi_ref[...]], axis=1)
    x = jnp.dot(h, w1_ref[...], preferred_element_type=jnp.float32) + b1_ref[...]
    h1 = jnp.where(x > 0, x, jnp.exp(x) - 1.0)
    t = jnp.dot(h1, w2_ref[...], preferred_element_type=jnp.float32) + b2_ref[...]
    temp = 1.0 / (1.0 + jnp.exp(-t))
    temp_ref[...] = jnp.transpose(temp).reshape(1, 1, ROWS)


_row_spec = pl.BlockSpec((ROWS, 256), lambda i: (i, 0))
_fix = lambda i: (0, 0)
_lane_spec = pl.BlockSpec((1, 1, ROWS), lambda i: (i, 0, 0))

_mlp_call = pl.pallas_call(
    _mlp_body,
    grid=(GRID,),
    in_specs=[
        _row_spec,
        _row_spec,
        pl.BlockSpec((512, 256), _fix),
        pl.BlockSpec((1, 256), _fix),
        pl.BlockSpec((256, 1), _fix),
        pl.BlockSpec((1, 1), _fix),
    ],
    out_specs=_lane_spec,
    out_shape=jax.ShapeDtypeStruct((GRID, 1, ROWS), jnp.float32),
)


def kernel(distance_to_center, edge_index, h_gru, gru_inp, beta, W1, b1, W2, b2):
    edge3 = edge_index.reshape(2, NROWS, 128).transpose(1, 0, 2)
    part = _segmin_kernel(distance_to_center, edge3)
    temp49 = _mlp_call(h_gru, gru_inp, W1, b1.reshape(1, 256), W2,
                       b2.reshape(1, 1))
    temp_flat = temp49.reshape(NPAD)
    ndw = jnp.broadcast_to((-(beta ** 2)).reshape(1), (16,))
    dist_s, switch_s, label_s = _merge_kernel(part, temp_flat, ndw)
    return (dist_s[:N], temp_flat[:N], switch_s[:N], label_s[:N])
